# baseline (speedup 1.0000x reference)
import functools

import jax
import jax.numpy as jnp
from jax import lax
from jax.experimental import pallas as pl
from jax.experimental.pallas import tpu as pltpu
from jax.experimental.pallas import tpu_sc as plsc

B = 16384
D = 64
NC = 2
NS = 16
NW = NC * NS
BW = B // NW
CHUNK = 128
NCH = BW // CHUNK


def _sc_body(user_hbm, item_hbm, rel_hbm, head_hbm, relidx_hbm, tail_hbm,
             out_hbm, hidx_v, ridx_v, tidx_v, hrow_v, rrow_v, trow_v,
             out_v, sem):
  wid = lax.axis_index("s") * NC + lax.axis_index("c")
  base = wid * BW

  for j in range(NCH):
    off = base + j * CHUNK
    pltpu.sync_copy(head_hbm.at[pl.ds(off, CHUNK)], hidx_v.at[j])
    pltpu.sync_copy(relidx_hbm.at[pl.ds(off, CHUNK)], ridx_v.at[j])
    pltpu.sync_copy(tail_hbm.at[pl.ds(off, CHUNK)], tidx_v.at[j])

  copies = []
  for j in range(NCH):
    dst = pl.ds(j * CHUNK, CHUNK)
    copies.append(pltpu.async_copy(user_hbm.at[hidx_v.at[j]],
                                   hrow_v.at[dst], sem))
    copies.append(pltpu.async_copy(rel_hbm.at[ridx_v.at[j]],
                                   rrow_v.at[dst], sem))
    copies.append(pltpu.async_copy(item_hbm.at[tidx_v.at[j]],
                                   trow_v.at[dst], sem))
  for c in copies:
    c.wait()

  lane = lax.iota(jnp.int32, 16)

  def grp(g, _):
    rows = g * 16 + lane

    def dstep(d, acc):
      cols = jnp.full((16,), d, jnp.int32)
      h = plsc.load_gather(hrow_v, [rows, cols])
      r = plsc.load_gather(rrow_v, [rows, cols])
      t = plsc.load_gather(trow_v, [rows, cols])
      return acc + jnp.abs(h + r - t)

    acc = lax.fori_loop(0, D, dstep, jnp.zeros((16,), jnp.float32))
    out_v[pl.ds(g * 16, 16)] = -acc
    return _

  lax.fori_loop(0, BW // 16, grp, None)

  pltpu.sync_copy(out_v, out_hbm.at[pl.ds(base, BW)])


@functools.partial(jax.jit, donate_argnums=())
def kernel(user_table, item_table, rel_table, head_idx, relation_idx,
           tail_idx):
  mesh = plsc.VectorSubcoreMesh(core_axis_name="c", subcore_axis_name="s")
  scores = pl.kernel(
      _sc_body,
      out_type=jax.ShapeDtypeStruct((B,), jnp.float32),
      mesh=mesh,
      compiler_params=pltpu.CompilerParams(
          needs_layout_passes=False, use_tc_tiling_on_sc=False),
      scratch_types=[
          pltpu.VMEM((NCH, CHUNK), jnp.int32),
          pltpu.VMEM((NCH, CHUNK), jnp.int32),
          pltpu.VMEM((NCH, CHUNK), jnp.int32),
          pltpu.VMEM((BW, D), jnp.float32),
          pltpu.VMEM((BW, D), jnp.float32),
          pltpu.VMEM((BW, D), jnp.float32),
          pltpu.VMEM((BW,), jnp.float32),
          pltpu.SemaphoreType.DMA,
      ],
  )(user_table, item_table, rel_table,
    head_idx.astype(jnp.int32), relation_idx.astype(jnp.int32),
    tail_idx.astype(jnp.int32))
  return scores
